# baseline (speedup 1.0000x reference)
import functools

import jax
import jax.numpy as jnp
from jax import lax
from jax.experimental import pallas as pl
from jax.experimental.pallas import tpu as pltpu
from jax.experimental.pallas import tpu_sc as plsc

N_NODES = 10000
D = 128
N_PAD = 10240
E_BLK = 128
NW = 32
N_SUB = 16
ROW_BLK = 1024
ROWS_PER_S = N_PAD // N_SUB


def _agg_body(h_hbm, pk_hbm, zeros_hbm, out_hbm,
              pk_v, sidx2, didx2, rows0, rows1, acc_sh, gsem0, gsem1):
    c = lax.axis_index("c")
    s = lax.axis_index("s")
    nblk = pk_hbm.shape[0] // NW
    w = s * 2 + c
    base = w * nblk
    pltpu.sync_copy(zeros_hbm.at[pl.ds(s * ROWS_PER_S, ROWS_PER_S)],
                    acc_sh.at[pl.ds(s * ROWS_PER_S, ROWS_PER_S)])
    pltpu.sync_copy(pk_hbm.at[pl.ds(base, nblk + 1)], pk_v)
    plsc.subcore_barrier()

    def unpack(b, slot):
        def u(k, carry):
            v = pk_v[b, 0, pl.ds(k * 16, 16)]
            sidx2[slot, 0, pl.ds(k * 16, 16)] = v & 0xFFFF
            didx2[slot, 0, pl.ds(k * 16, 16)] = v >> 16
            return carry
        lax.fori_loop(0, E_BLK // 16, u, 0)

    unpack(0, 0)
    pltpu.async_copy(h_hbm.at[sidx2.at[0, 0]], rows0, gsem0)

    def body(j, carry):
        b1 = 2 * j + 1
        b2 = 2 * j + 2
        unpack(b1, 1)
        pltpu.make_async_copy(h_hbm.at[sidx2.at[0, 0]], rows0, gsem0).wait()
        pltpu.async_copy(h_hbm.at[sidx2.at[1, 0]], rows1, gsem1)
        pltpu.sync_copy(rows0, acc_sh.at[didx2.at[0, 0]], add=True)
        unpack(b2, 0)
        pltpu.make_async_copy(h_hbm.at[sidx2.at[1, 0]], rows1, gsem1).wait()
        pltpu.async_copy(h_hbm.at[sidx2.at[0, 0]], rows0, gsem0)
        pltpu.sync_copy(rows1, acc_sh.at[didx2.at[1, 0]], add=True)
        return carry

    lax.fori_loop(0, nblk // 2, body, 0)
    pltpu.make_async_copy(h_hbm.at[sidx2.at[0, 0]], rows0, gsem0).wait()
    plsc.subcore_barrier()
    pltpu.sync_copy(acc_sh.at[pl.ds(s * ROWS_PER_S, ROWS_PER_S)],
                    out_hbm.at[c, pl.ds(s * ROWS_PER_S, ROWS_PER_S)])


def _deg_body(dst_hbm, ones_hbm, zeros_hbm, out_hbm, didx_v, ones_v, deg_sh):
    c = lax.axis_index("c")
    s = lax.axis_index("s")
    nblk = dst_hbm.shape[0] // NW
    w = s * 2 + c
    base = w * nblk
    pltpu.sync_copy(zeros_hbm.at[pl.ds(s * ROWS_PER_S, ROWS_PER_S)],
                    deg_sh.at[pl.ds(s * ROWS_PER_S, ROWS_PER_S)])
    pltpu.sync_copy(ones_hbm, ones_v)
    pltpu.sync_copy(dst_hbm.at[pl.ds(base, nblk)], didx_v)
    plsc.subcore_barrier()

    def body(i, carry):
        pltpu.sync_copy(ones_v, deg_sh.at[didx_v.at[i, 0]], add=True)
        return carry

    lax.fori_loop(0, nblk, body, 0)
    plsc.subcore_barrier()
    pltpu.sync_copy(deg_sh.at[pl.ds(s * ROWS_PER_S, ROWS_PER_S)],
                    out_hbm.at[c, pl.ds(s * ROWS_PER_S, ROWS_PER_S)])


def _sc_agg(h, pk2, zeros128):
    nblk = pk2.shape[0] // NW
    mesh = plsc.VectorSubcoreMesh(core_axis_name="c", subcore_axis_name="s")
    f = functools.partial(
        pl.kernel,
        out_type=jax.ShapeDtypeStruct((2, N_PAD, D), jnp.float32),
        mesh=mesh,
        scratch_types=[
            pltpu.VMEM((nblk + 1, 1, E_BLK), jnp.int32),
            pltpu.VMEM((2, 1, E_BLK), jnp.int32),
            pltpu.VMEM((2, 1, E_BLK), jnp.int32),
            pltpu.VMEM((E_BLK, D), jnp.float32),
            pltpu.VMEM((E_BLK, D), jnp.float32),
            pltpu.VMEM_SHARED((N_PAD, D), jnp.float32),
            pltpu.SemaphoreType.DMA,
            pltpu.SemaphoreType.DMA,
        ],
    )(_agg_body)
    return f(h, pk2, zeros128)


def _sc_deg(dst2, ones128, zeros128):
    nblk = dst2.shape[0] // NW
    mesh = plsc.VectorSubcoreMesh(core_axis_name="c", subcore_axis_name="s")
    f = functools.partial(
        pl.kernel,
        out_type=jax.ShapeDtypeStruct((2, N_PAD, D), jnp.float32),
        mesh=mesh,
        scratch_types=[
            pltpu.VMEM((nblk, 1, E_BLK), jnp.int32),
            pltpu.VMEM((E_BLK, D), jnp.float32),
            pltpu.VMEM_SHARED((N_PAD, D), jnp.float32),
        ],
    )(_deg_body)
    return f(dst2, ones128, zeros128)


def _mlp_in_body(h0_ref, w_ref, b_ref, o_ref):
    t = lax.dot_general(h0_ref[...], w_ref[...], (((1,), (1,)), ((), ())),
                        preferred_element_type=jnp.float32)
    o_ref[...] = jnp.tanh(t + b_ref[...])


def _tc_mlp_in(h0p, W_in, b_in):
    grid = (N_PAD // ROW_BLK,)
    return pl.pallas_call(
        _mlp_in_body,
        grid=grid,
        in_specs=[
            pl.BlockSpec((ROW_BLK, D), lambda i: (i, 0)),
            pl.BlockSpec((D, D), lambda i: (0, 0)),
            pl.BlockSpec((1, D), lambda i: (0, 0)),
        ],
        out_specs=pl.BlockSpec((ROW_BLK, D), lambda i: (i, 0)),
        out_shape=jax.ShapeDtypeStruct((N_PAD, D), jnp.float32),
    )(h0p, W_in, b_in.reshape(1, D))


def _layer_body(h_ref, a_ref, d_ref, ws_ref, bs_ref, wn_ref, o_ref):
    acc = a_ref[0] + a_ref[1]
    deg = d_ref[0, :, 0:1] + d_ref[1, :, 0:1]
    inv = 1.0 / jnp.maximum(deg, 1.0)
    neigh = acc * inv
    self_t = lax.dot_general(h_ref[...], ws_ref[...], (((1,), (1,)), ((), ())),
                             preferred_element_type=jnp.float32)
    nb_t = lax.dot_general(neigh, wn_ref[...], (((1,), (1,)), ((), ())),
                           preferred_element_type=jnp.float32)
    o_ref[...] = jnp.maximum(self_t + bs_ref[...] + nb_t, 0.0)


def _tc_layer(h, acc, degp, Ws, bs, Wn):
    grid = (N_PAD // ROW_BLK,)
    return pl.pallas_call(
        _layer_body,
        grid=grid,
        in_specs=[
            pl.BlockSpec((ROW_BLK, D), lambda i: (i, 0)),
            pl.BlockSpec((2, ROW_BLK, D), lambda i: (0, i, 0)),
            pl.BlockSpec((2, ROW_BLK, D), lambda i: (0, i, 0)),
            pl.BlockSpec((D, D), lambda i: (0, 0)),
            pl.BlockSpec((1, D), lambda i: (0, 0)),
            pl.BlockSpec((D, D), lambda i: (0, 0)),
        ],
        out_specs=pl.BlockSpec((ROW_BLK, D), lambda i: (i, 0)),
        out_shape=jax.ShapeDtypeStruct((N_PAD, D), jnp.float32),
    )(h, acc, degp, Ws, bs.reshape(1, D), Wn)


def kernel(h0, edge_index, W_in, b_in, W_self0, b_self0, W_neigh0,
           W_self1, b_self1, W_neigh1, W_self2, b_self2, W_neigh2):
    src = edge_index[0].astype(jnp.int32)
    dst = edge_index[1].astype(jnp.int32)
    e = src.shape[0]
    quantum = NW * E_BLK * 2
    e_pad = ((e + quantum - 1) // quantum) * quantum
    pad = e_pad - e
    srcp = jnp.concatenate([src, jnp.zeros((pad,), jnp.int32)])
    dstp = jnp.concatenate([dst, jnp.full((pad,), N_NODES, jnp.int32)])
    dst2 = dstp.reshape(-1, 1, E_BLK)
    pk = srcp | (dstp << 16)
    pk2 = jnp.concatenate([pk, jnp.full((E_BLK,), N_NODES << 16, jnp.int32)]).reshape(-1, 1, E_BLK)
    zeros128 = jnp.zeros((N_PAD, D), jnp.float32)
    ones128 = jnp.ones((E_BLK, D), jnp.float32)
    h0p = jnp.concatenate([h0, jnp.zeros((N_PAD - N_NODES, D), jnp.float32)], axis=0)

    degp = _sc_deg(dst2, ones128, zeros128)
    h = _tc_mlp_in(h0p, W_in, b_in)
    for Ws, bs, Wn in ((W_self0, b_self0, W_neigh0),
                       (W_self1, b_self1, W_neigh1),
                       (W_self2, b_self2, W_neigh2)):
        acc = _sc_agg(h, pk2, zeros128)
        h = _tc_layer(h, acc, degp, Ws, bs, Wn)
    return h[:N_NODES]


# speedup vs baseline: 1.4166x; 1.4166x over previous
import functools

import jax
import jax.numpy as jnp
from jax import lax
from jax.experimental import pallas as pl
from jax.experimental.pallas import tpu as pltpu
from jax.experimental.pallas import tpu_sc as plsc

N_NODES = 10000
D = 128
N_PAD = 10240
E_BLK = 128
NW = 32
N_SUB = 16
ROW_BLK = 1024
ROWS_PER_S = N_PAD // N_SUB


def _agg_body(h_hbm, src_hbm, dst_hbm, zeros_hbm, out_hbm,
              sidx_v, didx_v, rows_v, acc_sh, sem):
    c = lax.axis_index("c")
    s = lax.axis_index("s")
    nblk = src_hbm.shape[0] // NW
    w = s * 2 + c
    base = w * nblk
    pltpu.sync_copy(zeros_hbm.at[pl.ds(s * ROWS_PER_S, ROWS_PER_S)],
                    acc_sh.at[pl.ds(s * ROWS_PER_S, ROWS_PER_S)])
    pltpu.sync_copy(src_hbm.at[pl.ds(base, nblk)], sidx_v)
    pltpu.sync_copy(dst_hbm.at[pl.ds(base, nblk)], didx_v)
    plsc.subcore_barrier()

    def body(i, carry):
        pltpu.async_copy(h_hbm.at[sidx_v.at[i, 0]], rows_v, sem).wait()
        pltpu.sync_copy(rows_v, acc_sh.at[didx_v.at[i, 0]], add=True)
        return carry

    lax.fori_loop(0, nblk, body, 0)
    plsc.subcore_barrier()
    pltpu.sync_copy(acc_sh.at[pl.ds(s * ROWS_PER_S, ROWS_PER_S)],
                    out_hbm.at[c, pl.ds(s * ROWS_PER_S, ROWS_PER_S)])


def _deg_body(dst_hbm, ones_hbm, zeros_hbm, out_hbm, didx_v, ones_v, deg_sh):
    c = lax.axis_index("c")
    s = lax.axis_index("s")
    nblk = dst_hbm.shape[0] // NW
    w = s * 2 + c
    base = w * nblk
    pltpu.sync_copy(zeros_hbm.at[pl.ds(s * ROWS_PER_S, ROWS_PER_S)],
                    deg_sh.at[pl.ds(s * ROWS_PER_S, ROWS_PER_S)])
    pltpu.sync_copy(ones_hbm, ones_v)
    pltpu.sync_copy(dst_hbm.at[pl.ds(base, nblk)], didx_v)
    plsc.subcore_barrier()

    def body(i, carry):
        pltpu.sync_copy(ones_v, deg_sh.at[didx_v.at[i, 0]], add=True)
        return carry

    lax.fori_loop(0, nblk, body, 0)
    plsc.subcore_barrier()
    pltpu.sync_copy(deg_sh.at[pl.ds(s * ROWS_PER_S, ROWS_PER_S)],
                    out_hbm.at[c, pl.ds(s * ROWS_PER_S, ROWS_PER_S)])


def _sc_agg(h, src2, dst2, zeros128):
    nblk = src2.shape[0] // NW
    mesh = plsc.VectorSubcoreMesh(core_axis_name="c", subcore_axis_name="s")
    f = functools.partial(
        pl.kernel,
        out_type=jax.ShapeDtypeStruct((2, N_PAD, D), jnp.float32),
        mesh=mesh,
        scratch_types=[
            pltpu.VMEM((nblk, 1, E_BLK), jnp.int32),
            pltpu.VMEM((nblk, 1, E_BLK), jnp.int32),
            pltpu.VMEM((E_BLK, D), jnp.float32),
            pltpu.VMEM_SHARED((N_PAD, D), jnp.float32),
            pltpu.SemaphoreType.DMA,
        ],
    )(_agg_body)
    return f(h, src2, dst2, zeros128)


def _sc_deg(dst2, ones128, zeros128):
    nblk = dst2.shape[0] // NW
    mesh = plsc.VectorSubcoreMesh(core_axis_name="c", subcore_axis_name="s")
    f = functools.partial(
        pl.kernel,
        out_type=jax.ShapeDtypeStruct((2, N_PAD, D), jnp.float32),
        mesh=mesh,
        scratch_types=[
            pltpu.VMEM((nblk, 1, E_BLK), jnp.int32),
            pltpu.VMEM((E_BLK, D), jnp.float32),
            pltpu.VMEM_SHARED((N_PAD, D), jnp.float32),
        ],
    )(_deg_body)
    return f(dst2, ones128, zeros128)


def _mlp_in_body(h0_ref, w_ref, b_ref, o_ref):
    t = lax.dot_general(h0_ref[...], w_ref[...], (((1,), (1,)), ((), ())),
                        preferred_element_type=jnp.float32)
    o_ref[...] = jnp.tanh(t + b_ref[...])


def _tc_mlp_in(h0p, W_in, b_in):
    grid = (N_PAD // ROW_BLK,)
    return pl.pallas_call(
        _mlp_in_body,
        grid=grid,
        in_specs=[
            pl.BlockSpec((ROW_BLK, D), lambda i: (i, 0)),
            pl.BlockSpec((D, D), lambda i: (0, 0)),
            pl.BlockSpec((1, D), lambda i: (0, 0)),
        ],
        out_specs=pl.BlockSpec((ROW_BLK, D), lambda i: (i, 0)),
        out_shape=jax.ShapeDtypeStruct((N_PAD, D), jnp.float32),
    )(h0p, W_in, b_in.reshape(1, D))


def _layer_body(h_ref, a_ref, d_ref, ws_ref, bs_ref, wn_ref, o_ref):
    acc = a_ref[0] + a_ref[1]
    deg = d_ref[0, :, 0:1] + d_ref[1, :, 0:1]
    inv = 1.0 / jnp.maximum(deg, 1.0)
    neigh = acc * inv
    self_t = lax.dot_general(h_ref[...], ws_ref[...], (((1,), (1,)), ((), ())),
                             preferred_element_type=jnp.float32)
    nb_t = lax.dot_general(neigh, wn_ref[...], (((1,), (1,)), ((), ())),
                           preferred_element_type=jnp.float32)
    o_ref[...] = jnp.maximum(self_t + bs_ref[...] + nb_t, 0.0)


def _tc_layer(h, acc, degp, Ws, bs, Wn):
    grid = (N_PAD // ROW_BLK,)
    return pl.pallas_call(
        _layer_body,
        grid=grid,
        in_specs=[
            pl.BlockSpec((ROW_BLK, D), lambda i: (i, 0)),
            pl.BlockSpec((2, ROW_BLK, D), lambda i: (0, i, 0)),
            pl.BlockSpec((2, ROW_BLK, D), lambda i: (0, i, 0)),
            pl.BlockSpec((D, D), lambda i: (0, 0)),
            pl.BlockSpec((1, D), lambda i: (0, 0)),
            pl.BlockSpec((D, D), lambda i: (0, 0)),
        ],
        out_specs=pl.BlockSpec((ROW_BLK, D), lambda i: (i, 0)),
        out_shape=jax.ShapeDtypeStruct((N_PAD, D), jnp.float32),
    )(h, acc, degp, Ws, bs.reshape(1, D), Wn)


def kernel(h0, edge_index, W_in, b_in, W_self0, b_self0, W_neigh0,
           W_self1, b_self1, W_neigh1, W_self2, b_self2, W_neigh2):
    src = edge_index[0].astype(jnp.int32)
    dst = edge_index[1].astype(jnp.int32)
    e = src.shape[0]
    quantum = NW * E_BLK
    e_pad = ((e + quantum - 1) // quantum) * quantum
    pad = e_pad - e
    src2 = jnp.concatenate([src, jnp.zeros((pad,), jnp.int32)]).reshape(-1, 1, E_BLK)
    dst2 = jnp.concatenate([dst, jnp.full((pad,), N_NODES, jnp.int32)]).reshape(-1, 1, E_BLK)
    zeros128 = jnp.zeros((N_PAD, D), jnp.float32)
    ones128 = jnp.ones((E_BLK, D), jnp.float32)
    h0p = jnp.concatenate([h0, jnp.zeros((N_PAD - N_NODES, D), jnp.float32)], axis=0)

    degp = _sc_deg(dst2, ones128, zeros128)
    h = _tc_mlp_in(h0p, W_in, b_in)
    for Ws, bs, Wn in ((W_self0, b_self0, W_neigh0),
                       (W_self1, b_self1, W_neigh1),
                       (W_self2, b_self2, W_neigh2)):
        acc = _sc_agg(h, src2, dst2, zeros128)
        h = _tc_layer(h, acc, degp, Ws, bs, Wn)
    return h[:N_NODES]
